# Initial kernel scaffold; baseline (speedup 1.0000x reference)
import functools

import jax
import jax.numpy as jnp
from jax import lax
from jax.experimental import pallas as pl
from jax.experimental.pallas import tpu as pltpu
from jax.experimental.pallas import tpu_sc as plsc

NEG_SLOPE = 0.2

NC = 2
NS = 16
NW = NC * NS

F32 = jnp.float32


def _proj_body(x_ref, w_ref, hs_ref, hd_ref, hv_ref):
    p = lax.dot_general(
        x_ref[...], w_ref[...], (((1,), (1,)), ((), ())),
        preferred_element_type=F32)
    hs_ref[...] = p[:, 0:128]
    hd_ref[...] = p[:, 128:256]
    hv_ref[...] = p[:, 256:384]


def _projections(x, wcat, n_blk):
    n = x.shape[0]
    grid = n // n_blk
    out = jax.ShapeDtypeStruct((n, 128), F32)
    return pl.pallas_call(
        _proj_body,
        grid=(grid,),
        in_specs=[
            pl.BlockSpec((n_blk, 128), lambda i: (i, 0)),
            pl.BlockSpec((384, 128), lambda i: (0, 0)),
        ],
        out_specs=[
            pl.BlockSpec((n_blk, 128), lambda i: (i, 0)),
            pl.BlockSpec((n_blk, 128), lambda i: (i, 0)),
            pl.BlockSpec((n_blk, 128), lambda i: (i, 0)),
        ],
        out_shape=[out, out, out],
    )(x, wcat)


def _make_edge_sum(E, chunk):
    epw = E // NW
    nchunk = epw // chunk
    mesh = plsc.VectorSubcoreMesh(core_axis_name="c", subcore_axis_name="s")

    @functools.partial(
        pl.kernel,
        out_type=jax.ShapeDtypeStruct((E, 128), F32),
        mesh=mesh,
        scratch_types=[
            pltpu.VMEM((chunk,), jnp.int32),
            pltpu.VMEM((chunk,), jnp.int32),
            pltpu.VMEM((chunk, 128), F32),
            pltpu.VMEM((chunk, 128), F32),
            pltpu.SemaphoreType.DMA,
            pltpu.SemaphoreType.DMA,
        ],
    )
    def edge_sum(hs_hbm, hd_hbm, src_hbm, dst_hbm, ssum_hbm,
                 sidx, didx, hi, hj, sem1, sem2):
        wid = lax.axis_index("s") * NC + lax.axis_index("c")
        base = wid * epw

        def chunk_body(j, carry):
            eb = base + j * chunk
            pltpu.sync_copy(src_hbm.at[pl.ds(eb, chunk)], sidx)
            pltpu.sync_copy(dst_hbm.at[pl.ds(eb, chunk)], didx)
            cp1 = pltpu.async_copy(hs_hbm.at[sidx], hi, sem1)
            cp2 = pltpu.async_copy(hd_hbm.at[didx], hj, sem2)
            cp1.wait()
            cp2.wait()

            def row_body(ci, c2):
                for t in range(8):
                    sl = pl.ds(t * 16, 16)
                    hi[ci, sl] = hi[ci, sl] + hj[ci, sl]
                return c2

            lax.fori_loop(0, chunk, row_body, 0)
            pltpu.sync_copy(hi, ssum_hbm.at[pl.ds(eb, chunk)])
            return carry

        lax.fori_loop(0, nchunk, chunk_body, 0)

    return edge_sum


def _logits_body(s_ref, a_ref, lg_ref, gm_ref):
    i = pl.program_id(0)
    s = s_ref[...]
    t = jnp.where(s >= 0.0, s, NEG_SLOPE * s)
    lg = lax.dot_general(
        a_ref[...], t, (((0,), (1,)), ((), ())), preferred_element_type=F32)
    lg_ref[...] = lg
    bm = jnp.broadcast_to(jnp.max(lg, axis=1, keepdims=True), (8, 128))

    @pl.when(i == 0)
    def _():
        gm_ref[...] = bm

    @pl.when(i > 0)
    def _():
        gm_ref[...] = jnp.maximum(gm_ref[...], bm)


def _logits(ssum, att8, e_blk):
    E = ssum.shape[0]
    grid = E // e_blk
    return pl.pallas_call(
        _logits_body,
        grid=(grid,),
        in_specs=[
            pl.BlockSpec((e_blk, 128), lambda i: (i, 0)),
            pl.BlockSpec((128, 8), lambda i: (0, 0)),
        ],
        out_specs=[
            pl.BlockSpec((8, e_blk), lambda i: (0, i)),
            pl.BlockSpec((8, 128), lambda i: (0, 0)),
        ],
        out_shape=[
            jax.ShapeDtypeStruct((8, E), F32),
            jax.ShapeDtypeStruct((8, 128), F32),
        ],
    )(ssum, att8)


def _make_edge_accum(N, E, chunk):
    epw = E // NW
    nchunk = epw // chunk
    rows_per_tile = N // NS
    mesh = plsc.VectorSubcoreMesh(core_axis_name="c", subcore_axis_name="s")

    zslices = []
    off = 0
    while off < rows_per_tile:
        sz = min(chunk, rows_per_tile - off)
        zslices.append((off, sz))
        off += sz

    @functools.partial(
        pl.kernel,
        out_type=(
            jax.ShapeDtypeStruct((NC, N, 16), F32),
            jax.ShapeDtypeStruct((NC, N, 128), F32),
        ),
        mesh=mesh,
        scratch_types=[
            pltpu.VMEM((chunk,), jnp.int32),
            pltpu.VMEM((chunk,), jnp.int32),
            pltpu.VMEM((4, chunk), F32),
            pltpu.VMEM((chunk, 16), F32),
            pltpu.VMEM((chunk, 128), F32),
            pltpu.VMEM((8, 128), F32),
            pltpu.VMEM_SHARED((N, 16), F32),
            pltpu.VMEM_SHARED((N, 128), F32),
            pltpu.SemaphoreType.DMA,
        ],
    )
    def edge_accum(src_hbm, dst_hbm, lg_hbm, gm_hbm, val_hbm,
                   den_hbm, out_hbm,
                   sidx, didx, lv, exv, vals, gmv, den_sp, acc_sp, sem):
        cc = lax.axis_index("c")
        ss = lax.axis_index("s")
        wid = ss * NC + cc
        base = wid * epw
        rb = ss * rows_per_tile

        z16 = jnp.zeros((16,), F32)

        def zero_body(ci, carry):
            exv[ci, :] = z16
            for t in range(8):
                vals[ci, pl.ds(t * 16, 16)] = z16
            return carry

        lax.fori_loop(0, chunk, zero_body, 0)

        for off, sz in zslices:
            pltpu.sync_copy(exv.at[pl.ds(0, sz)],
                            den_sp.at[pl.ds(rb + off, sz)])
            pltpu.sync_copy(vals.at[pl.ds(0, sz)],
                            acc_sp.at[pl.ds(rb + off, sz)])
        plsc.subcore_barrier()

        pltpu.sync_copy(gm_hbm, gmv)
        iota = lax.iota(jnp.int32, 16)
        hcols = [jnp.full((16,), h, jnp.int32) for h in range(4)]
        kvec = chunk // 16

        def chunk_body(j, carry):
            eb = base + j * chunk
            pltpu.sync_copy(src_hbm.at[pl.ds(eb, chunk)], sidx)
            pltpu.sync_copy(dst_hbm.at[pl.ds(eb, chunk)], didx)
            cp = pltpu.async_copy(val_hbm.at[sidx], vals, sem)
            for h in range(4):
                pltpu.sync_copy(lg_hbm.at[h, pl.ds(eb, chunk)], lv.at[h])
            for h in range(4):
                g = gmv[h, 0]
                for k in range(kvec):
                    v = jnp.exp(lv[h, pl.ds(k * 16, 16)] - g)
                    plsc.store_scatter(exv, [k * 16 + iota, hcols[h]], v)
            pltpu.sync_copy(exv, den_sp.at[didx], add=True)
            cp.wait()

            def scale_body(ci, c2):
                for h in range(4):
                    w = exv[ci, h]
                    for t in range(2):
                        sl = pl.ds(h * 32 + t * 16, 16)
                        vals[ci, sl] = vals[ci, sl] * w
                return c2

            lax.fori_loop(0, chunk, scale_body, 0)
            pltpu.sync_copy(vals, acc_sp.at[didx], add=True)
            return carry

        lax.fori_loop(0, nchunk, chunk_body, 0)
        plsc.subcore_barrier()

        for off, sz in zslices:
            pltpu.sync_copy(den_sp.at[pl.ds(rb + off, sz)],
                            den_hbm.at[cc, pl.ds(rb + off, sz)])
            pltpu.sync_copy(acc_sp.at[pl.ds(rb + off, sz)],
                            out_hbm.at[cc, pl.ds(rb + off, sz)])

    return edge_accum


def _final_body(den_ref, acc_ref, r_ref, b_ref, o_ref):
    den = den_ref[0] + den_ref[1]
    acc = acc_ref[0] + acc_ref[1]
    dexp = lax.dot_general(
        den, r_ref[...], (((1,), (0,)), ((), ())), preferred_element_type=F32)
    dexp = jnp.where(dexp == 0.0, 1.0, dexp)
    o_ref[...] = acc / dexp + b_ref[...]


def _finalize(den_part, acc_part, rmat, bias2d, n_blk):
    n = den_part.shape[1]
    grid = n // n_blk
    return pl.pallas_call(
        _final_body,
        grid=(grid,),
        in_specs=[
            pl.BlockSpec((2, n_blk, 16), lambda i: (0, i, 0)),
            pl.BlockSpec((2, n_blk, 128), lambda i: (0, i, 0)),
            pl.BlockSpec((16, 128), lambda i: (0, 0)),
            pl.BlockSpec((1, 128), lambda i: (0, 0)),
        ],
        out_specs=pl.BlockSpec((n_blk, 128), lambda i: (i, 0)),
        out_shape=jax.ShapeDtypeStruct((n, 128), F32),
    )(den_part, acc_part, rmat, bias2d)


def kernel(x, edge_index, W, W1, W2, att, bias):
    n, in_f = x.shape
    E = edge_index.shape[1]
    heads, out_f = att.shape[1], att.shape[2]
    hf = heads * out_f

    src = edge_index[0]
    dst = edge_index[1]
    wcat = jnp.concatenate([W1, W2, W], axis=0)

    cols = jnp.repeat(jnp.arange(heads), out_f)
    att8 = jnp.zeros((hf, 8), F32).at[jnp.arange(hf), cols].set(
        att.reshape(hf).astype(F32))
    rmat = jnp.zeros((16, hf), F32).at[cols, jnp.arange(hf)].set(1.0)
    bias2d = bias.reshape(1, hf).astype(F32)

    h_src, h_dst, values = _projections(x, wcat, 1000)
    ssum = _make_edge_sum(E, 200)(h_src, h_dst, src, dst)
    lgT8, gmax8 = _logits(ssum, att8, 640)
    den_part, acc_part = _make_edge_accum(n, E, 80)(
        src, dst, lgT8, gmax8, values)
    return _finalize(den_part, acc_part, rmat, bias2d, 1000)


# speedup vs baseline: 21.8524x; 21.8524x over previous
import functools

import numpy as _np

import jax
import jax.numpy as jnp
from jax import lax
from jax.experimental import pallas as pl
from jax.experimental.pallas import tpu as pltpu
from jax.experimental.pallas import tpu_sc as plsc

NEG_SLOPE = 0.2

NC = 2
NS = 16
NW = NC * NS

F32 = jnp.float32


def _proj_body(x_ref, w_ref, hs_ref, hd_ref, hv_ref):
    p = lax.dot_general(
        x_ref[...], w_ref[...], (((1,), (1,)), ((), ())),
        preferred_element_type=F32, precision=lax.Precision.HIGHEST)
    hs_ref[...] = p[:, 0:128]
    hd_ref[...] = p[:, 128:256]
    hv_ref[...] = p[:, 256:384]


def _projections(x, wcat, n_blk):
    n = x.shape[0]
    grid = n // n_blk
    out = jax.ShapeDtypeStruct((n, 128), F32)
    return pl.pallas_call(
        _proj_body,
        grid=(grid,),
        in_specs=[
            pl.BlockSpec((n_blk, 128), lambda i: (i, 0)),
            pl.BlockSpec((384, 128), lambda i: (0, 0)),
        ],
        out_specs=[
            pl.BlockSpec((n_blk, 128), lambda i: (i, 0)),
            pl.BlockSpec((n_blk, 128), lambda i: (i, 0)),
            pl.BlockSpec((n_blk, 128), lambda i: (i, 0)),
        ],
        out_shape=[out, out, out],
    )(x, wcat)


def _make_edge_sum(E):
    chunk = 128
    nblk = E // chunk
    nfull = nblk // NW
    tail = nblk % NW
    mesh = plsc.VectorSubcoreMesh(core_axis_name="c", subcore_axis_name="s")

    @functools.partial(
        pl.kernel,
        out_type=jax.ShapeDtypeStruct((E, 128), F32),
        mesh=mesh,
        scratch_types=[
            pltpu.VMEM((chunk,), jnp.int32),
            pltpu.VMEM((chunk,), jnp.int32),
            pltpu.VMEM((chunk, 128), F32),
            pltpu.VMEM((chunk, 128), F32),
            pltpu.SemaphoreType.DMA,
            pltpu.SemaphoreType.DMA,
        ],
    )
    def edge_sum(hs_hbm, hd_hbm, src_hbm, dst_hbm, ssum_hbm,
                 sidx, didx, hi, hj, sem1, sem2):
        wid = lax.axis_index("s") * NC + lax.axis_index("c")

        def process(eb):
            pltpu.sync_copy(src_hbm.at[pl.ds(eb, chunk)], sidx)
            pltpu.sync_copy(dst_hbm.at[pl.ds(eb, chunk)], didx)
            cp1 = pltpu.async_copy(hs_hbm.at[sidx], hi, sem1)
            cp2 = pltpu.async_copy(hd_hbm.at[didx], hj, sem2)
            cp1.wait()
            cp2.wait()

            def row_body(ci, c2):
                for t in range(8):
                    sl = pl.ds(t * 16, 16)
                    hi[ci, sl] = hi[ci, sl] + hj[ci, sl]
                return c2

            lax.fori_loop(0, chunk, row_body, 0)
            pltpu.sync_copy(hi, ssum_hbm.at[pl.ds(eb, chunk)])

        def chunk_body(j, carry):
            process((j * NW + wid) * chunk)
            return carry

        lax.fori_loop(0, nfull, chunk_body, 0)
        if tail:
            @pl.when(wid < tail)
            def _():
                process((nfull * NW + wid) * chunk)

    return edge_sum


def _logits_body(s_ref, a_ref, lg_ref, gm_ref):
    i = pl.program_id(0)
    s = s_ref[...]
    t = jnp.where(s >= 0.0, s, NEG_SLOPE * s)
    lg = lax.dot_general(
        a_ref[...], t, (((0,), (1,)), ((), ())), preferred_element_type=F32,
        precision=lax.Precision.HIGHEST)
    lg_ref[...] = lg
    bm = jnp.broadcast_to(jnp.max(lg, axis=1, keepdims=True), (8, 128))

    @pl.when(i == 0)
    def _():
        gm_ref[...] = bm

    @pl.when(i > 0)
    def _():
        gm_ref[...] = jnp.maximum(gm_ref[...], bm)


def _logits(ssum, att8, e_blk):
    E = ssum.shape[0]
    grid = E // e_blk
    return pl.pallas_call(
        _logits_body,
        grid=(grid,),
        in_specs=[
            pl.BlockSpec((e_blk, 128), lambda i: (i, 0)),
            pl.BlockSpec((128, 8), lambda i: (0, 0)),
        ],
        out_specs=[
            pl.BlockSpec((8, e_blk), lambda i: (0, i)),
            pl.BlockSpec((8, 128), lambda i: (0, 0)),
        ],
        out_shape=[
            jax.ShapeDtypeStruct((8, E), F32),
            jax.ShapeDtypeStruct((8, 128), F32),
        ],
    )(ssum, att8)


def _make_edge_accum(N, E):
    chunk = 128
    nblk = E // chunk
    nfull = nblk // NW
    tail = nblk % NW
    rows_per_tile = 640
    n_pad = rows_per_tile * NS
    assert n_pad >= N
    mesh = plsc.VectorSubcoreMesh(core_axis_name="c", subcore_axis_name="s")

    @functools.partial(
        pl.kernel,
        out_type=(
            jax.ShapeDtypeStruct((NC, 4 * n_pad), F32),
            jax.ShapeDtypeStruct((NC, n_pad, 128), F32),
        ),
        mesh=mesh,
        scratch_types=[
            pltpu.VMEM((chunk,), jnp.int32),
            pltpu.VMEM((chunk,), jnp.int32),
            pltpu.VMEM((chunk,), jnp.int32),
            pltpu.VMEM((8, chunk), F32),
            pltpu.VMEM((8 * chunk,), F32),
            pltpu.VMEM((chunk, 128), F32),
            pltpu.VMEM((8, 128), F32),
            pltpu.VMEM_SHARED((4 * n_pad,), F32),
            pltpu.VMEM_SHARED((n_pad, 128), F32),
            pltpu.SemaphoreType.DMA,
        ],
    )
    def edge_accum(src_hbm, dst_hbm, lg_hbm, gm_hbm, val_hbm,
                   den_hbm, out_hbm,
                   sidx, didx, didxh, lv, exh, vals, gmv, den_sp, acc_sp, sem):
        cc = lax.axis_index("c")
        ss = lax.axis_index("s")
        wid = ss * NC + cc
        rb = ss * rows_per_tile

        z16 = jnp.zeros((16,), F32)

        def zero_body(ci, carry):
            for t in range(8):
                vals[ci, pl.ds(t * 16, 16)] = z16
            return carry

        lax.fori_loop(0, chunk, zero_body, 0)
        for k in range(8 * chunk // 16):
            exh[pl.ds(k * 16, 16)] = z16

        for h in range(4):
            pltpu.sync_copy(exh.at[pl.ds(0, rows_per_tile)],
                            den_sp.at[pl.ds(h * n_pad + rb, rows_per_tile)])
        for j in range(rows_per_tile // chunk):
            pltpu.sync_copy(vals, acc_sp.at[pl.ds(rb + j * chunk, chunk)])
        plsc.subcore_barrier()

        pltpu.sync_copy(gm_hbm, gmv)
        kvec = chunk // 16

        def process(eb):
            pltpu.sync_copy(src_hbm.at[pl.ds(eb, chunk)], sidx)
            pltpu.sync_copy(dst_hbm.at[pl.ds(eb, chunk)], didx)
            cp = pltpu.async_copy(val_hbm.at[sidx], vals, sem)
            pltpu.sync_copy(lg_hbm.at[pl.ds(0, 8), pl.ds(eb, chunk)], lv)
            for h in range(4):
                g = gmv[h, pl.ds(0, 16)][0]
                for k in range(kvec):
                    exh[pl.ds(h * chunk + k * 16, 16)] = jnp.exp(
                        lv[h, pl.ds(k * 16, 16)] - g)
            for h in range(4):
                for k in range(kvec):
                    sl = pl.ds(k * 16, 16)
                    didxh[sl] = didx[sl] + (h * n_pad)
                pltpu.sync_copy(exh.at[pl.ds(h * chunk, chunk)],
                                den_sp.at[didxh], add=True)
            cp.wait()

            def scale_body(ci, c2):
                for h in range(4):
                    w = exh[pl.ds(h * chunk + ci, 16)][0]
                    for t in range(2):
                        sl = pl.ds(h * 32 + t * 16, 16)
                        vals[ci, sl] = vals[ci, sl] * w
                return c2

            lax.fori_loop(0, chunk, scale_body, 0)
            pltpu.sync_copy(vals, acc_sp.at[didx], add=True)

        def chunk_body(j, carry):
            process((j * NW + wid) * chunk)
            return carry

        lax.fori_loop(0, nfull, chunk_body, 0)
        if tail:
            @pl.when(wid < tail)
            def _():
                process((nfull * NW + wid) * chunk)

        plsc.subcore_barrier()

        for h in range(4):
            dsl = pl.ds(h * n_pad + rb, rows_per_tile)
            pltpu.sync_copy(den_sp.at[dsl], den_hbm.at[cc, dsl])
        for j in range(rows_per_tile // chunk):
            sl = pl.ds(rb + j * chunk, chunk)
            pltpu.sync_copy(acc_sp.at[sl], out_hbm.at[cc, sl])

    return edge_accum


def _final_body(den_ref, acc_ref, r_ref, b_ref, o_ref):
    den = den_ref[0] + den_ref[1]
    acc = acc_ref[0] + acc_ref[1]
    den = jnp.where(den == 0.0, 1.0, den)
    recip = 1.0 / den
    rexp = lax.dot_general(
        recip, r_ref[...], (((0,), (0,)), ((), ())),
        preferred_element_type=F32,
        precision=lax.Precision.HIGHEST)
    o_ref[...] = acc * rexp + b_ref[...]


def _finalize(den_part, acc_part, rmat4, bias2d, n_pad, n_blk):
    grid = n_pad // n_blk
    return pl.pallas_call(
        _final_body,
        grid=(grid,),
        in_specs=[
            pl.BlockSpec((2, 4, n_blk), lambda i: (0, 0, i)),
            pl.BlockSpec((2, n_blk, 128), lambda i: (0, i, 0)),
            pl.BlockSpec((4, 128), lambda i: (0, 0)),
            pl.BlockSpec((1, 128), lambda i: (0, 0)),
        ],
        out_specs=pl.BlockSpec((n_blk, 128), lambda i: (i, 0)),
        out_shape=jax.ShapeDtypeStruct((n_pad, 128), F32),
    )(den_part, acc_part, rmat4, bias2d)


def kernel(x, edge_index, W, W1, W2, att, bias):
    n, in_f = x.shape
    E = edge_index.shape[1]
    heads, out_f = att.shape[1], att.shape[2]
    hf = heads * out_f

    src = edge_index[0]
    dst = edge_index[1]
    wcat = jnp.concatenate([W1, W2, W], axis=0)

    eye8 = jnp.asarray(_np.eye(heads, 8, dtype=_np.float32))
    att8 = (att.reshape(heads, out_f, 1).astype(F32)
            * eye8[:, None, :]).reshape(hf, 8)
    rmat4 = jnp.asarray(_np.repeat(
        _np.eye(heads, dtype=_np.float32), out_f, axis=1))
    bias2d = bias.reshape(1, hf).astype(F32)

    h_src, h_dst, values = _projections(x, wcat, 1000)
    ssum = _make_edge_sum(E)(h_src, h_dst, src, dst)
    lgT8, gmax8 = _logits(ssum, att8, 640)
    den_part, acc_part = _make_edge_accum(n, E)(
        src, dst, lgT8, gmax8, values)
    n_pad = acc_part.shape[1]
    den_part = den_part.reshape(2, heads, n_pad)
    out = _finalize(den_part, acc_part, rmat4, bias2d, n_pad, 640)
    return out[:n]
